# baseline (device time: 104619 ns/iter reference)
import functools

import jax
import jax.numpy as jnp
from jax import lax
from jax.experimental import pallas as pl
from jax.experimental.pallas import tpu as pltpu

N_DEV = 4
SQ = 2048
SKV = 2048
DM = 1024
HL = 8
DH = 128
DL = HL * DH
QB = 256
KW = 512
WIN = 128
QROWS = SQ // N_DEV
SCALE = 0.08838834764831843


def kernel(x, Wq, K_ext, V_ext, Wo):
    my = lax.axis_index("i")
    xb = x[0].astype(jnp.bfloat16)
    wq_my = lax.dynamic_slice_in_dim(Wq, my * DL, DL, axis=1)
    wq_my = wq_my.astype(jnp.bfloat16)
    wo_my = lax.dynamic_slice_in_dim(Wo, my * DL, DL, axis=0)
    wo_my = wo_my.astype(jnp.bfloat16)
    kb = K_ext[0].transpose(1, 0, 2).astype(jnp.bfloat16)
    vb = V_ext[0].transpose(1, 0, 2).astype(jnp.bfloat16)

    def body(x_ref, wq_ref, k_ref, v_ref, wo_ref, out_ref,
             q_ref, ctx_ref,
             dr_send, dr_recv, ag_sendR, ag_recvR, ag_sendL, ag_recvL,
             drs_sems, drr_sems,
             sendR_sems, recvR_sems, sendL_sems, recvL_sems):
        my_pos = lax.axis_index("i")
        left = lax.rem(my_pos + N_DEV - 1, N_DEV)
        right = lax.rem(my_pos + 1, N_DEV)
        diag = lax.rem(my_pos + 2, N_DEV)

        barrier_sem = pltpu.get_barrier_semaphore()
        for nbr in (left, right, diag):
            pl.semaphore_signal(barrier_sem, inc=1, device_id=(nbr,),
                                device_id_type=pl.DeviceIdType.MESH)
        pl.semaphore_wait(barrier_sem, 3)

        def compute_quarter(qtr):
            for j in range(QROWS // QB):
                row = qtr * QROWS + j * QB
                rows = pl.ds(row, QB)
                q_ref[rows, :] = lax.dot_general(
                    x_ref[rows, :], wq_ref[...], (((1,), (0,)), ((), ())),
                    preferred_element_type=jnp.float32).astype(jnp.bfloat16)
                s = jnp.minimum(jnp.maximum(row - 128, 0), SKV - KW)
                s = (s // 128) * 128
                qi = lax.broadcasted_iota(jnp.int32, (QB, KW), 0) + row
                kj = lax.broadcasted_iota(jnp.int32, (QB, KW), 1) + s
                mask = jnp.abs(qi - kj) <= WIN
                for h in range(HL):
                    hcols = pl.ds(h * DH, DH)
                    qblk = q_ref[rows, hcols]
                    kwin = k_ref[h, pl.ds(s, KW), :]
                    vwin = v_ref[h, pl.ds(s, KW), :]
                    scores = lax.dot_general(
                        qblk, kwin, (((1,), (1,)), ((), ())),
                        preferred_element_type=jnp.float32) * SCALE
                    scores = jnp.where(mask, scores, -1e9)
                    w = jnp.exp(scores)
                    recip = 1.0 / jnp.sum(w, axis=1, keepdims=True)
                    ctx_blk = lax.dot_general(
                        w.astype(jnp.bfloat16), vwin, (((1,), (0,)), ((), ())),
                        preferred_element_type=jnp.float32) * recip
                    ctx_ref[rows, hcols] = ctx_blk.astype(jnp.bfloat16)
                out_ref[0, rows, :] = lax.dot_general(
                    ctx_ref[rows, :], wo_ref[...], (((1,), (0,)), ((), ())),
                    preferred_element_type=jnp.float32)

        def qmod(c):
            return lax.rem(my_pos + c + 2 * N_DEV, N_DEV)

        def rowQ(q):
            return pl.ds(q * QROWS, QROWS)

        def rowA(q):
            return pl.ds(q * QROWS, QB)

        def rowB(q):
            return pl.ds(q * QROWS + QB, QB)

        dr_rdmas = []
        for r in (1, 2, 3):
            qtr = qmod(r)
            compute_quarter(qtr)
            dr_send[r - 1] = out_ref[0, rowQ(qtr), :].astype(jnp.bfloat16)
            rdma = pltpu.make_async_remote_copy(
                src_ref=dr_send.at[r - 1],
                dst_ref=dr_recv.at[3 - r],
                send_sem=drs_sems.at[r - 1],
                recv_sem=drr_sems.at[3 - r],
                device_id=(qmod(r),),
                device_id_type=pl.DeviceIdType.MESH,
            )
            rdma.start()
            dr_rdmas.append(rdma)

        compute_quarter(qmod(0))
        for rdma in dr_rdmas:
            rdma.wait_send()
        for s in range(3):
            pltpu.make_async_remote_copy(
                src_ref=dr_send.at[s], dst_ref=dr_recv.at[s],
                send_sem=drs_sems.at[s], recv_sem=drr_sems.at[s],
                device_id=(right,), device_id_type=pl.DeviceIdType.MESH,
            ).wait_recv()
        for j in range(QROWS // QB):
            rows = pl.ds(qmod(0) * QROWS + j * QB, QB)
            srows = pl.ds(j * QB, QB)
            acc = out_ref[0, rows, :]
            for s in range(3):
                acc = acc + dr_recv[s, srows, :].astype(jnp.float32)
            out_ref[0, rows, :] = acc

        ag_sendR[0] = out_ref[0, rowA(qmod(0)), :].astype(jnp.bfloat16)
        ag_sendL[0] = out_ref[0, rowB(qmod(0)), :].astype(jnp.bfloat16)

        def copy(src, dst, s_sems, r_sems, idx, dev):
            return pltpu.make_async_remote_copy(
                src_ref=src, dst_ref=dst,
                send_sem=s_sems.at[idx], recv_sem=r_sems.at[idx],
                device_id=(dev,), device_id_type=pl.DeviceIdType.MESH)

        for t in range(N_DEV - 1):
            srcR = ag_sendR.at[0] if t == 0 else ag_recvR.at[t - 1]
            srcL = ag_sendL.at[0] if t == 0 else ag_recvL.at[t - 1]
            aR = copy(srcR, ag_recvR.at[t], sendR_sems, recvR_sems, t, right)
            aL = copy(srcL, ag_recvL.at[t], sendL_sems, recvL_sems, t, left)
            aR.start()
            aL.start()
            aR.wait()
            aL.wait()
            out_ref[0, rowA(qmod(-1 - t)), :] = ag_recvR[t].astype(jnp.float32)
            out_ref[0, rowB(qmod(1 + t)), :] = ag_recvL[t].astype(jnp.float32)

        @functools.partial(pl.run_scoped, sem2=pltpu.SemaphoreType.REGULAR)
        def _(sem2):
            for nbr in (left, right, diag):
                pl.semaphore_signal(sem2, inc=1, device_id=(nbr,),
                                    device_id_type=pl.DeviceIdType.MESH)
            pl.semaphore_wait(sem2, 3)

    return pl.pallas_call(
        body,
        out_shape=jax.ShapeDtypeStruct((1, SQ, DM), jnp.float32),
        in_specs=[pl.BlockSpec(memory_space=pltpu.VMEM)] * 5,
        out_specs=pl.BlockSpec(memory_space=pltpu.VMEM),
        scratch_shapes=[
            pltpu.VMEM((SQ, DL), jnp.bfloat16),
            pltpu.VMEM((SQ, DL), jnp.bfloat16),
            pltpu.VMEM((3, QROWS, DM), jnp.bfloat16),
            pltpu.VMEM((3, QROWS, DM), jnp.bfloat16),
            pltpu.VMEM((1, QB, DM), jnp.bfloat16),
            pltpu.VMEM((3, QB, DM), jnp.bfloat16),
            pltpu.VMEM((1, QB, DM), jnp.bfloat16),
            pltpu.VMEM((3, QB, DM), jnp.bfloat16),
            pltpu.SemaphoreType.DMA((3,)),
            pltpu.SemaphoreType.DMA((3,)),
            pltpu.SemaphoreType.DMA((3,)),
            pltpu.SemaphoreType.DMA((3,)),
            pltpu.SemaphoreType.DMA((3,)),
            pltpu.SemaphoreType.DMA((3,)),
        ],
        compiler_params=pltpu.CompilerParams(
            collective_id=0,
            vmem_limit_bytes=100 * 1024 * 1024,
        ),
    )(xb, wq_my, kb, vb, wo_my)


# device time: 99986 ns/iter; 1.0463x vs baseline; 1.0463x over previous
import functools

import jax
import jax.numpy as jnp
from jax import lax
from jax.experimental import pallas as pl
from jax.experimental.pallas import tpu as pltpu

N_DEV = 4
SQ = 2048
SKV = 2048
DM = 1024
HL = 8
DH = 128
DL = HL * DH
QB = 256
KW = 512
WIN = 128
QROWS = SQ // N_DEV
SCALE = 0.08838834764831843
QSCALE = SCALE * 1.4426950408889634
HQB = 128


def kernel(x, Wq, K_ext, V_ext, Wo):
    my = lax.axis_index("i")
    xb = x[0].astype(jnp.bfloat16)
    wq_my = lax.dynamic_slice_in_dim(Wq, my * DL, DL, axis=1)
    wq_my = wq_my.astype(jnp.bfloat16)
    wo_my = lax.dynamic_slice_in_dim(Wo, my * DL, DL, axis=0)
    wo_my = wo_my.astype(jnp.bfloat16)
    kb = K_ext[0].transpose(1, 0, 2).astype(jnp.bfloat16)
    vb = V_ext[0].transpose(1, 0, 2).astype(jnp.bfloat16)

    def body(x_ref, wq_ref, k_ref, v_ref, wo_ref, out_ref,
             q_ref, ctx_ref,
             dr_send, dr_recv, ag_sendR, ag_recvR, ag_sendL, ag_recvL,
             drs_sems, drr_sems,
             sendR_sems, recvR_sems, sendL_sems, recvL_sems):
        my_pos = lax.axis_index("i")
        left = lax.rem(my_pos + N_DEV - 1, N_DEV)
        right = lax.rem(my_pos + 1, N_DEV)
        diag = lax.rem(my_pos + 2, N_DEV)

        barrier_sem = pltpu.get_barrier_semaphore()
        for nbr in (left, right, diag):
            pl.semaphore_signal(barrier_sem, inc=1, device_id=(nbr,),
                                device_id_type=pl.DeviceIdType.MESH)
        pl.semaphore_wait(barrier_sem, 3)

        def compute_quarter(qtr):
            for j in range(QROWS // QB):
                row = qtr * QROWS + j * QB
                rows = pl.ds(row, QB)
                q_ref[rows, :] = (lax.dot_general(
                    x_ref[rows, :], wq_ref[...], (((1,), (0,)), ((), ())),
                    preferred_element_type=jnp.float32)
                    * QSCALE).astype(jnp.bfloat16)
                s = jnp.minimum(jnp.maximum(row - 128, 0), SKV - KW)
                s = (s // 128) * 128
                qi = lax.broadcasted_iota(jnp.int32, (QB, KW), 0) + row
                kj = lax.broadcasted_iota(jnp.int32, (QB, KW), 1) + s
                maskbias = jnp.where(jnp.abs(qi - kj) <= WIN, 0.0, -1e30)
                for h in range(HL):
                    hcols = pl.ds(h * DH, DH)
                    qblk = q_ref[rows, hcols]
                    kwin = k_ref[h, pl.ds(s, KW), :]
                    vwin = v_ref[h, pl.ds(s, KW), :]
                    scores = lax.dot_general(
                        qblk, kwin, (((1,), (1,)), ((), ())),
                        preferred_element_type=jnp.float32)
                    w = jnp.exp2(scores + maskbias)
                    recip = 1.0 / jnp.sum(w, axis=1, keepdims=True)
                    ctx_blk = lax.dot_general(
                        w.astype(jnp.bfloat16), vwin, (((1,), (0,)), ((), ())),
                        preferred_element_type=jnp.float32) * recip
                    ctx_ref[rows, hcols] = ctx_blk.astype(jnp.bfloat16)
                out_ref[0, rows, :] = lax.dot_general(
                    ctx_ref[rows, :], wo_ref[...], (((1,), (0,)), ((), ())),
                    preferred_element_type=jnp.float32)

        def qmod(c):
            return lax.rem(my_pos + c + 2 * N_DEV, N_DEV)

        def rowQ(q):
            return pl.ds(q * QROWS, QROWS)

        def rowA(q):
            return pl.ds(q * QROWS, QB)

        def rowB(q):
            return pl.ds(q * QROWS + QB, QB)

        dr_rdmas = []
        for r in (1, 2, 3):
            qtr = qmod(r)
            compute_quarter(qtr)
            dr_send[r - 1] = out_ref[0, rowQ(qtr), :].astype(jnp.bfloat16)
            rdma = pltpu.make_async_remote_copy(
                src_ref=dr_send.at[r - 1],
                dst_ref=dr_recv.at[3 - r],
                send_sem=drs_sems.at[r - 1],
                recv_sem=drr_sems.at[3 - r],
                device_id=(qmod(r),),
                device_id_type=pl.DeviceIdType.MESH,
            )
            rdma.start()
            dr_rdmas.append(rdma)

        compute_quarter(qmod(0))
        for rdma in dr_rdmas:
            rdma.wait_send()
        for s in range(3):
            pltpu.make_async_remote_copy(
                src_ref=dr_send.at[s], dst_ref=dr_recv.at[s],
                send_sem=drs_sems.at[s], recv_sem=drr_sems.at[s],
                device_id=(right,), device_id_type=pl.DeviceIdType.MESH,
            ).wait_recv()
        for j in range(QROWS // QB):
            rows = pl.ds(qmod(0) * QROWS + j * QB, QB)
            srows = pl.ds(j * QB, QB)
            acc = out_ref[0, rows, :]
            for s in range(3):
                acc = acc + dr_recv[s, srows, :].astype(jnp.float32)
            out_ref[0, rows, :] = acc

        def copy(src, dst, s_sems, r_sems, idx, dev):
            return pltpu.make_async_remote_copy(
                src_ref=src, dst_ref=dst,
                send_sem=s_sems.at[idx], recv_sem=r_sems.at[idx],
                device_id=(dev,), device_id_type=pl.DeviceIdType.MESH)

        started = []
        for u in range(2):
            baseA = qmod(0) * QROWS + u * HQB
            baseB = qmod(0) * QROWS + QB + u * HQB
            ag_sendR[u] = out_ref[0, pl.ds(baseA, HQB), :].astype(jnp.bfloat16)
            ag_sendL[u] = out_ref[0, pl.ds(baseB, HQB), :].astype(jnp.bfloat16)
        for u in range(2):
            dR = copy(ag_sendR.at[u], ag_recvR.at[0, u],
                      sendR_sems, recvR_sems, u, right)
            dL = copy(ag_sendL.at[u], ag_recvL.at[0, u],
                      sendL_sems, recvL_sems, u, left)
            dR.start()
            dL.start()
            started += [dR, dL]
        for t in range(N_DEV - 1):
            qR = qmod(-1 - t)
            qL = qmod(1 + t)
            for u in range(2):
                copy(ag_sendR.at[u], ag_recvR.at[t, u],
                     sendR_sems, recvR_sems, 2 * t + u, right).wait_recv()
                copy(ag_sendL.at[u], ag_recvL.at[t, u],
                     sendL_sems, recvL_sems, 2 * t + u, left).wait_recv()
                if t < N_DEV - 2:
                    fR = copy(ag_recvR.at[t, u], ag_recvR.at[t + 1, u],
                              sendR_sems, recvR_sems, 2 * (t + 1) + u, right)
                    fL = copy(ag_recvL.at[t, u], ag_recvL.at[t + 1, u],
                              sendL_sems, recvL_sems, 2 * (t + 1) + u, left)
                    fR.start()
                    fL.start()
                    started += [fR, fL]
                out_ref[0, pl.ds(qR * QROWS + u * HQB, HQB), :] = (
                    ag_recvR[t, u].astype(jnp.float32))
                out_ref[0, pl.ds(qL * QROWS + QB + u * HQB, HQB), :] = (
                    ag_recvL[t, u].astype(jnp.float32))
        for d in started:
            d.wait_send()

        @functools.partial(pl.run_scoped, sem2=pltpu.SemaphoreType.REGULAR)
        def _(sem2):
            for nbr in (left, right, diag):
                pl.semaphore_signal(sem2, inc=1, device_id=(nbr,),
                                    device_id_type=pl.DeviceIdType.MESH)
            pl.semaphore_wait(sem2, 3)

    return pl.pallas_call(
        body,
        out_shape=jax.ShapeDtypeStruct((1, SQ, DM), jnp.float32),
        in_specs=[pl.BlockSpec(memory_space=pltpu.VMEM)] * 5,
        out_specs=pl.BlockSpec(memory_space=pltpu.VMEM),
        scratch_shapes=[
            pltpu.VMEM((SQ, DL), jnp.bfloat16),
            pltpu.VMEM((SQ, DL), jnp.bfloat16),
            pltpu.VMEM((3, QROWS, DM), jnp.bfloat16),
            pltpu.VMEM((3, QROWS, DM), jnp.bfloat16),
            pltpu.VMEM((2, HQB, DM), jnp.bfloat16),
            pltpu.VMEM((3, 2, HQB, DM), jnp.bfloat16),
            pltpu.VMEM((2, HQB, DM), jnp.bfloat16),
            pltpu.VMEM((3, 2, HQB, DM), jnp.bfloat16),
            pltpu.SemaphoreType.DMA((3,)),
            pltpu.SemaphoreType.DMA((3,)),
            pltpu.SemaphoreType.DMA((6,)),
            pltpu.SemaphoreType.DMA((6,)),
            pltpu.SemaphoreType.DMA((6,)),
            pltpu.SemaphoreType.DMA((6,)),
        ],
        compiler_params=pltpu.CompilerParams(
            collective_id=0,
            vmem_limit_bytes=100 * 1024 * 1024,
        ),
    )(xb, wq_my, kb, vb, wo_my)


# device time: 98527 ns/iter; 1.0618x vs baseline; 1.0148x over previous
import functools

import jax
import jax.numpy as jnp
from jax import lax
from jax.experimental import pallas as pl
from jax.experimental.pallas import tpu as pltpu

N_DEV = 4
SQ = 2048
SKV = 2048
DM = 1024
HL = 8
DH = 128
DL = HL * DH
QB = 256
KW = 512
WIN = 128
QROWS = SQ // N_DEV
SCALE = 0.08838834764831843
QSCALE = SCALE * 1.4426950408889634
HQB = 128


def kernel(x, Wq, K_ext, V_ext, Wo):
    xb = x[0]
    kb = K_ext[0].reshape(SKV, DL)
    vb = V_ext[0].reshape(SKV, DL)

    def body(x_hbm, wq_hbm, k_hbm, v_hbm, wo_hbm, out_ref,
             stag, x_ref, k_ref, v_ref, wq_ref, wo_ref,
             q_ref, ctx_ref,
             dr_send, dr_recv, ag_sendR, ag_recvR, ag_sendL, ag_recvL,
             copy_sems, drs_sems, drr_sems,
             sendR_sems, recvR_sems, sendL_sems, recvL_sems):
        my_pos = lax.axis_index("i")
        left = lax.rem(my_pos + N_DEV - 1, N_DEV)
        right = lax.rem(my_pos + 1, N_DEV)
        diag = lax.rem(my_pos + 2, N_DEV)

        barrier_sem = pltpu.get_barrier_semaphore()
        for nbr in (left, right, diag):
            pl.semaphore_signal(barrier_sem, inc=1, device_id=(nbr,),
                                device_id_type=pl.DeviceIdType.MESH)
        pl.semaphore_wait(barrier_sem, 3)

        CH = 1024
        srcs = [
            wq_hbm.at[:, pl.ds(my_pos * DL, DL)],
            wo_hbm.at[pl.ds(my_pos * DL, DL), :],
            x_hbm.at[pl.ds(0, CH), :],
            x_hbm.at[pl.ds(CH, CH), :],
            k_hbm.at[pl.ds(0, CH), :],
            k_hbm.at[pl.ds(CH, CH), :],
            v_hbm.at[pl.ds(0, CH), :],
            v_hbm.at[pl.ds(CH, CH), :],
        ]

        def store_chunk(i, slot):
            val = stag[slot].astype(jnp.bfloat16)
            if i == 0:
                wq_ref[...] = val
            elif i == 1:
                wo_ref[...] = val
            elif i in (2, 3):
                x_ref[pl.ds((i - 2) * CH, CH), :] = val
            elif i in (4, 5):
                k_ref[pl.ds((i - 4) * CH, CH), :] = val
            else:
                v_ref[pl.ds((i - 6) * CH, CH), :] = val

        dmas = [None, None]
        for i in range(2):
            dmas[i] = pltpu.make_async_copy(srcs[i], stag.at[i],
                                            copy_sems.at[i])
            dmas[i].start()
        for i in range(len(srcs)):
            slot = i % 2
            dmas[slot].wait()
            store_chunk(i, slot)
            if i + 2 < len(srcs):
                d = pltpu.make_async_copy(srcs[i + 2], stag.at[slot],
                                          copy_sems.at[slot])
                d.start()
                dmas[slot] = d

        def compute_quarter(qtr):
            for j in range(QROWS // QB):
                row = qtr * QROWS + j * QB
                rows = pl.ds(row, QB)
                q_ref[rows, :] = (lax.dot_general(
                    x_ref[rows, :], wq_ref[...], (((1,), (0,)), ((), ())),
                    preferred_element_type=jnp.float32)
                    * QSCALE).astype(jnp.bfloat16)
                s = jnp.minimum(jnp.maximum(row - 128, 0), SKV - KW)
                s = (s // 128) * 128
                qi = lax.broadcasted_iota(jnp.int32, (QB, KW), 0) + row
                kj = lax.broadcasted_iota(jnp.int32, (QB, KW), 1) + s
                maskbias = jnp.where(jnp.abs(qi - kj) <= WIN, 0.0, -1e30)
                for h in range(HL):
                    hcols = pl.ds(h * DH, DH)
                    qblk = q_ref[rows, hcols]
                    kwin = k_ref[pl.ds(s, KW), hcols]
                    vwin = v_ref[pl.ds(s, KW), hcols]
                    scores = lax.dot_general(
                        qblk, kwin, (((1,), (1,)), ((), ())),
                        preferred_element_type=jnp.float32)
                    w = jnp.exp2(scores + maskbias)
                    recip = 1.0 / jnp.sum(w, axis=1, keepdims=True)
                    ctx_blk = lax.dot_general(
                        w.astype(jnp.bfloat16), vwin, (((1,), (0,)), ((), ())),
                        preferred_element_type=jnp.float32) * recip
                    ctx_ref[rows, hcols] = ctx_blk.astype(jnp.bfloat16)
                out_ref[0, rows, :] = lax.dot_general(
                    ctx_ref[rows, :], wo_ref[...], (((1,), (0,)), ((), ())),
                    preferred_element_type=jnp.float32)

        def qmod(c):
            return lax.rem(my_pos + c + 2 * N_DEV, N_DEV)

        def rowQ(q):
            return pl.ds(q * QROWS, QROWS)

        def rowA(q):
            return pl.ds(q * QROWS, QB)

        def rowB(q):
            return pl.ds(q * QROWS + QB, QB)

        dr_rdmas = []
        for r in (1, 2, 3):
            qtr = qmod(r)
            compute_quarter(qtr)
            dr_send[r - 1] = out_ref[0, rowQ(qtr), :].astype(jnp.bfloat16)
            rdma = pltpu.make_async_remote_copy(
                src_ref=dr_send.at[r - 1],
                dst_ref=dr_recv.at[3 - r],
                send_sem=drs_sems.at[r - 1],
                recv_sem=drr_sems.at[3 - r],
                device_id=(qmod(r),),
                device_id_type=pl.DeviceIdType.MESH,
            )
            rdma.start()
            dr_rdmas.append(rdma)

        compute_quarter(qmod(0))
        for rdma in dr_rdmas:
            rdma.wait_send()
        for s in range(3):
            pltpu.make_async_remote_copy(
                src_ref=dr_send.at[s], dst_ref=dr_recv.at[s],
                send_sem=drs_sems.at[s], recv_sem=drr_sems.at[s],
                device_id=(right,), device_id_type=pl.DeviceIdType.MESH,
            ).wait_recv()
        for j in range(QROWS // QB):
            rows = pl.ds(qmod(0) * QROWS + j * QB, QB)
            srows = pl.ds(j * QB, QB)
            acc = out_ref[0, rows, :]
            for s in range(3):
                acc = acc + dr_recv[s, srows, :].astype(jnp.float32)
            out_ref[0, rows, :] = acc

        def copy(src, dst, s_sems, r_sems, idx, dev):
            return pltpu.make_async_remote_copy(
                src_ref=src, dst_ref=dst,
                send_sem=s_sems.at[idx], recv_sem=r_sems.at[idx],
                device_id=(dev,), device_id_type=pl.DeviceIdType.MESH)

        started = []
        for u in range(2):
            baseA = qmod(0) * QROWS + u * HQB
            baseB = qmod(0) * QROWS + QB + u * HQB
            ag_sendR[u] = out_ref[0, pl.ds(baseA, HQB), :].astype(jnp.bfloat16)
            ag_sendL[u] = out_ref[0, pl.ds(baseB, HQB), :].astype(jnp.bfloat16)
        for u in range(2):
            dR = copy(ag_sendR.at[u], ag_recvR.at[0, u],
                      sendR_sems, recvR_sems, u, right)
            dL = copy(ag_sendL.at[u], ag_recvL.at[0, u],
                      sendL_sems, recvL_sems, u, left)
            dR.start()
            dL.start()
            started += [dR, dL]
        for t in range(N_DEV - 1):
            qR = qmod(-1 - t)
            qL = qmod(1 + t)
            for u in range(2):
                copy(ag_sendR.at[u], ag_recvR.at[t, u],
                     sendR_sems, recvR_sems, 2 * t + u, right).wait_recv()
                copy(ag_sendL.at[u], ag_recvL.at[t, u],
                     sendL_sems, recvL_sems, 2 * t + u, left).wait_recv()
                if t < N_DEV - 2:
                    fR = copy(ag_recvR.at[t, u], ag_recvR.at[t + 1, u],
                              sendR_sems, recvR_sems, 2 * (t + 1) + u, right)
                    fL = copy(ag_recvL.at[t, u], ag_recvL.at[t + 1, u],
                              sendL_sems, recvL_sems, 2 * (t + 1) + u, left)
                    fR.start()
                    fL.start()
                    started += [fR, fL]
                out_ref[0, pl.ds(qR * QROWS + u * HQB, HQB), :] = (
                    ag_recvR[t, u].astype(jnp.float32))
                out_ref[0, pl.ds(qL * QROWS + QB + u * HQB, HQB), :] = (
                    ag_recvL[t, u].astype(jnp.float32))
        for d in started:
            d.wait_send()

        @functools.partial(pl.run_scoped, sem2=pltpu.SemaphoreType.REGULAR)
        def _(sem2):
            for nbr in (left, right, diag):
                pl.semaphore_signal(sem2, inc=1, device_id=(nbr,),
                                    device_id_type=pl.DeviceIdType.MESH)
            pl.semaphore_wait(sem2, 3)

    return pl.pallas_call(
        body,
        out_shape=jax.ShapeDtypeStruct((1, SQ, DM), jnp.float32),
        in_specs=[pl.BlockSpec(memory_space=pl.ANY)] * 5,
        out_specs=pl.BlockSpec(memory_space=pltpu.VMEM),
        scratch_shapes=[
            pltpu.VMEM((2, 1024, DM), jnp.float32),
            pltpu.VMEM((SQ, DM), jnp.bfloat16),
            pltpu.VMEM((SKV, DL), jnp.bfloat16),
            pltpu.VMEM((SKV, DL), jnp.bfloat16),
            pltpu.VMEM((DM, DL), jnp.bfloat16),
            pltpu.VMEM((DL, DM), jnp.bfloat16),
            pltpu.VMEM((SQ, DL), jnp.bfloat16),
            pltpu.VMEM((SQ, DL), jnp.bfloat16),
            pltpu.VMEM((3, QROWS, DM), jnp.bfloat16),
            pltpu.VMEM((3, QROWS, DM), jnp.bfloat16),
            pltpu.VMEM((2, HQB, DM), jnp.bfloat16),
            pltpu.VMEM((3, 2, HQB, DM), jnp.bfloat16),
            pltpu.VMEM((2, HQB, DM), jnp.bfloat16),
            pltpu.VMEM((3, 2, HQB, DM), jnp.bfloat16),
            pltpu.SemaphoreType.DMA((2,)),
            pltpu.SemaphoreType.DMA((3,)),
            pltpu.SemaphoreType.DMA((3,)),
            pltpu.SemaphoreType.DMA((6,)),
            pltpu.SemaphoreType.DMA((6,)),
            pltpu.SemaphoreType.DMA((6,)),
            pltpu.SemaphoreType.DMA((6,)),
        ],
        compiler_params=pltpu.CompilerParams(
            collective_id=0,
            vmem_limit_bytes=100 * 1024 * 1024,
        ),
    )(xb, Wq, kb, vb, Wo)


# device time: 97664 ns/iter; 1.0712x vs baseline; 1.0088x over previous
import functools

import jax
import jax.numpy as jnp
from jax import lax
from jax.experimental import pallas as pl
from jax.experimental.pallas import tpu as pltpu

N_DEV = 4
SQ = 2048
SKV = 2048
DM = 1024
HL = 8
DH = 128
DL = HL * DH
QB = 256
KW = 512
WIN = 128
QROWS = SQ // N_DEV
SCALE = 0.08838834764831843
QSCALE = SCALE * 1.4426950408889634
HQB = 128


def kernel(x, Wq, K_ext, V_ext, Wo):
    xb = x[0]
    kb = K_ext[0].reshape(SKV, DL)
    vb = V_ext[0].reshape(SKV, DL)

    def body(x_hbm, wq_hbm, k_hbm, v_hbm, wo_hbm, out_ref,
             stag, x_ref, k_ref, v_ref, wq_ref, wo_ref,
             q_ref, ctx_ref,
             dr_send, dr_recv, ag_sendR, ag_recvR, ag_sendL, ag_recvL,
             copy_sems, drs_sems, drr_sems,
             sendR_sems, recvR_sems, sendL_sems, recvL_sems):
        my_pos = lax.axis_index("i")
        left = lax.rem(my_pos + N_DEV - 1, N_DEV)
        right = lax.rem(my_pos + 1, N_DEV)
        diag = lax.rem(my_pos + 2, N_DEV)

        barrier_sem = pltpu.get_barrier_semaphore()
        for nbr in (left, right, diag):
            pl.semaphore_signal(barrier_sem, inc=1, device_id=(nbr,),
                                device_id_type=pl.DeviceIdType.MESH)

        CH = 1024
        srcs = [
            wq_hbm.at[:, pl.ds(my_pos * DL, DL)],
            x_hbm.at[pl.ds(0, CH), :],
            x_hbm.at[pl.ds(CH, CH), :],
            k_hbm.at[pl.ds(0, CH), :],
            k_hbm.at[pl.ds(CH, CH), :],
            v_hbm.at[pl.ds(0, CH), :],
            v_hbm.at[pl.ds(CH, CH), :],
            wo_hbm.at[pl.ds(my_pos * DL, DL), :],
        ]

        def store_chunk(i, slot):
            val = stag[slot].astype(jnp.bfloat16)
            if i == 0:
                wq_ref[...] = val
            elif i in (1, 2):
                x_ref[pl.ds((i - 1) * CH, CH), :] = val
            elif i in (3, 4):
                k_ref[pl.ds((i - 3) * CH, CH), :] = val
            elif i in (5, 6):
                v_ref[pl.ds((i - 5) * CH, CH), :] = val
            else:
                wo_ref[...] = val

        dmas = [None, None]

        def start_dma(i):
            d = pltpu.make_async_copy(srcs[i], stag.at[i % 2],
                                      copy_sems.at[i % 2])
            d.start()
            dmas[i % 2] = d

        def finish_dma(i):
            dmas[i % 2].wait()
            store_chunk(i, i % 2)

        start_dma(0)
        start_dma(1)
        pl.semaphore_wait(barrier_sem, 3)
        for i in range(3):
            finish_dma(i)
            if i + 2 < len(srcs):
                start_dma(i + 2)
        for j8 in range(SQ // QB):
            rows = pl.ds(j8 * QB, QB)
            q_ref[rows, :] = (lax.dot_general(
                x_ref[rows, :], wq_ref[...], (((1,), (0,)), ((), ())),
                preferred_element_type=jnp.float32)
                * QSCALE).astype(jnp.bfloat16)
            if j8 % 2 == 1:
                i = 3 + j8 // 2
                finish_dma(i)
                if i + 2 < len(srcs):
                    start_dma(i + 2)
        finish_dma(7)

        def compute_quarter(qtr):
            for j in range(QROWS // QB):
                row = qtr * QROWS + j * QB
                rows = pl.ds(row, QB)
                s = jnp.minimum(jnp.maximum(row - 128, 0), SKV - KW)
                s = (s // 128) * 128
                qi = lax.broadcasted_iota(jnp.int32, (QB, KW), 0) + row
                kj = lax.broadcasted_iota(jnp.int32, (QB, KW), 1) + s
                maskbias = jnp.where(jnp.abs(qi - kj) <= WIN, 0.0, -1e30)
                for h in range(HL):
                    hcols = pl.ds(h * DH, DH)
                    qblk = q_ref[rows, hcols]
                    kwin = k_ref[pl.ds(s, KW), hcols]
                    vwin = v_ref[pl.ds(s, KW), hcols]
                    scores = lax.dot_general(
                        qblk, kwin, (((1,), (1,)), ((), ())),
                        preferred_element_type=jnp.float32)
                    w = jnp.exp2(scores + maskbias)
                    recip = 1.0 / jnp.sum(w, axis=1, keepdims=True)
                    ctx_blk = lax.dot_general(
                        w.astype(jnp.bfloat16), vwin, (((1,), (0,)), ((), ())),
                        preferred_element_type=jnp.float32) * recip
                    ctx_ref[rows, hcols] = ctx_blk.astype(jnp.bfloat16)
                out_ref[0, rows, :] = lax.dot_general(
                    ctx_ref[rows, :], wo_ref[...], (((1,), (0,)), ((), ())),
                    preferred_element_type=jnp.float32)

        def qmod(c):
            return lax.rem(my_pos + c + 2 * N_DEV, N_DEV)

        def rowQ(q):
            return pl.ds(q * QROWS, QROWS)

        def rowA(q):
            return pl.ds(q * QROWS, QB)

        def rowB(q):
            return pl.ds(q * QROWS + QB, QB)

        dr_rdmas = []
        for r in (1, 2, 3):
            qtr = qmod(r)
            compute_quarter(qtr)
            dr_send[r - 1] = out_ref[0, rowQ(qtr), :].astype(jnp.bfloat16)
            rdma = pltpu.make_async_remote_copy(
                src_ref=dr_send.at[r - 1],
                dst_ref=dr_recv.at[3 - r],
                send_sem=drs_sems.at[r - 1],
                recv_sem=drr_sems.at[3 - r],
                device_id=(qmod(r),),
                device_id_type=pl.DeviceIdType.MESH,
            )
            rdma.start()
            dr_rdmas.append(rdma)

        compute_quarter(qmod(0))
        for rdma in dr_rdmas:
            rdma.wait_send()
        for s in range(3):
            pltpu.make_async_remote_copy(
                src_ref=dr_send.at[s], dst_ref=dr_recv.at[s],
                send_sem=drs_sems.at[s], recv_sem=drr_sems.at[s],
                device_id=(right,), device_id_type=pl.DeviceIdType.MESH,
            ).wait_recv()
        for j in range(QROWS // QB):
            rows = pl.ds(qmod(0) * QROWS + j * QB, QB)
            srows = pl.ds(j * QB, QB)
            acc = out_ref[0, rows, :]
            for s in range(3):
                acc = acc + dr_recv[s, srows, :].astype(jnp.float32)
            out_ref[0, rows, :] = acc

        def copy(src, dst, s_sems, r_sems, idx, dev):
            return pltpu.make_async_remote_copy(
                src_ref=src, dst_ref=dst,
                send_sem=s_sems.at[idx], recv_sem=r_sems.at[idx],
                device_id=(dev,), device_id_type=pl.DeviceIdType.MESH)

        started = []
        for u in range(2):
            baseA = qmod(0) * QROWS + u * HQB
            baseB = qmod(0) * QROWS + QB + u * HQB
            ag_sendR[u] = out_ref[0, pl.ds(baseA, HQB), :].astype(jnp.bfloat16)
            ag_sendL[u] = out_ref[0, pl.ds(baseB, HQB), :].astype(jnp.bfloat16)
        for u in range(2):
            dR = copy(ag_sendR.at[u], ag_recvR.at[0, u],
                      sendR_sems, recvR_sems, u, right)
            dL = copy(ag_sendL.at[u], ag_recvL.at[0, u],
                      sendL_sems, recvL_sems, u, left)
            dR.start()
            dL.start()
            started += [dR, dL]
        for t in range(N_DEV - 1):
            qR = qmod(-1 - t)
            qL = qmod(1 + t)
            for u in range(2):
                copy(ag_sendR.at[u], ag_recvR.at[t, u],
                     sendR_sems, recvR_sems, 2 * t + u, right).wait_recv()
                copy(ag_sendL.at[u], ag_recvL.at[t, u],
                     sendL_sems, recvL_sems, 2 * t + u, left).wait_recv()
                if t < N_DEV - 2:
                    fR = copy(ag_recvR.at[t, u], ag_recvR.at[t + 1, u],
                              sendR_sems, recvR_sems, 2 * (t + 1) + u, right)
                    fL = copy(ag_recvL.at[t, u], ag_recvL.at[t + 1, u],
                              sendL_sems, recvL_sems, 2 * (t + 1) + u, left)
                    fR.start()
                    fL.start()
                    started += [fR, fL]
                out_ref[0, pl.ds(qR * QROWS + u * HQB, HQB), :] = (
                    ag_recvR[t, u].astype(jnp.float32))
                out_ref[0, pl.ds(qL * QROWS + QB + u * HQB, HQB), :] = (
                    ag_recvL[t, u].astype(jnp.float32))
        for d in started:
            d.wait_send()

        @functools.partial(pl.run_scoped, sem2=pltpu.SemaphoreType.REGULAR)
        def _(sem2):
            for nbr in (left, right, diag):
                pl.semaphore_signal(sem2, inc=1, device_id=(nbr,),
                                    device_id_type=pl.DeviceIdType.MESH)
            pl.semaphore_wait(sem2, 3)

    return pl.pallas_call(
        body,
        out_shape=jax.ShapeDtypeStruct((1, SQ, DM), jnp.float32),
        in_specs=[pl.BlockSpec(memory_space=pl.ANY)] * 5,
        out_specs=pl.BlockSpec(memory_space=pltpu.VMEM),
        scratch_shapes=[
            pltpu.VMEM((2, 1024, DM), jnp.float32),
            pltpu.VMEM((SQ, DM), jnp.bfloat16),
            pltpu.VMEM((SKV, DL), jnp.bfloat16),
            pltpu.VMEM((SKV, DL), jnp.bfloat16),
            pltpu.VMEM((DM, DL), jnp.bfloat16),
            pltpu.VMEM((DL, DM), jnp.bfloat16),
            pltpu.VMEM((SQ, DL), jnp.bfloat16),
            pltpu.VMEM((SQ, DL), jnp.bfloat16),
            pltpu.VMEM((3, QROWS, DM), jnp.bfloat16),
            pltpu.VMEM((3, QROWS, DM), jnp.bfloat16),
            pltpu.VMEM((2, HQB, DM), jnp.bfloat16),
            pltpu.VMEM((3, 2, HQB, DM), jnp.bfloat16),
            pltpu.VMEM((2, HQB, DM), jnp.bfloat16),
            pltpu.VMEM((3, 2, HQB, DM), jnp.bfloat16),
            pltpu.SemaphoreType.DMA((2,)),
            pltpu.SemaphoreType.DMA((3,)),
            pltpu.SemaphoreType.DMA((3,)),
            pltpu.SemaphoreType.DMA((6,)),
            pltpu.SemaphoreType.DMA((6,)),
            pltpu.SemaphoreType.DMA((6,)),
            pltpu.SemaphoreType.DMA((6,)),
        ],
        compiler_params=pltpu.CompilerParams(
            collective_id=0,
            vmem_limit_bytes=100 * 1024 * 1024,
        ),
    )(xb, Wq, kb, vb, Wo)


# device time: 89386 ns/iter; 1.1704x vs baseline; 1.0926x over previous
import functools

import jax
import jax.numpy as jnp
from jax import lax
from jax.experimental import pallas as pl
from jax.experimental.pallas import tpu as pltpu

N_DEV = 4
SQ = 2048
SKV = 2048
DM = 1024
HL = 8
DH = 128
DL = HL * DH
QB = 256
KW = 512
WIN = 128
QROWS = SQ // N_DEV
SCALE = 0.08838834764831843
QSCALE = SCALE * 1.4426950408889634
HQB = 128


def kernel(x, Wq, K_ext, V_ext, Wo):
    xb = x[0]
    kb = K_ext[0]
    vb = V_ext[0]

    def body(x_hbm, wq_hbm, k_hbm, v_hbm, wo_hbm, out_ref,
             stagA, stagB, x_ref, k_ref, v_ref, wq_ref, wo_ref,
             q_ref, ctx_ref,
             dr_send, dr_recv, ag_sendR, ag_recvR, ag_sendL, ag_recvL,
             copyA_sems, copyB_sems, drs_sems, drr_sems,
             sendR_sems, recvR_sems, sendL_sems, recvL_sems):
        my_pos = lax.axis_index("i")
        left = lax.rem(my_pos + N_DEV - 1, N_DEV)
        right = lax.rem(my_pos + 1, N_DEV)
        diag = lax.rem(my_pos + 2, N_DEV)

        barrier_sem = pltpu.get_barrier_semaphore()
        for nbr in (left, right, diag):
            pl.semaphore_signal(barrier_sem, inc=1, device_id=(nbr,),
                                device_id_type=pl.DeviceIdType.MESH)

        CH = 1024
        srcsA = [
            wq_hbm.at[:, pl.ds(my_pos * DL, DL)],
            x_hbm.at[pl.ds(0, CH), :],
            x_hbm.at[pl.ds(CH, CH), :],
            wo_hbm.at[pl.ds(my_pos * DL, DL), :],
        ]

        def storeA(i, slot):
            val = stagA[slot].astype(jnp.bfloat16)
            if i == 0:
                wq_ref[...] = val
            elif i in (1, 2):
                x_ref[pl.ds((i - 1) * CH, CH), :] = val
            else:
                wo_ref[...] = val

        dmasA = [None, None]
        dmasB = [None, None]

        def startA(i):
            d = pltpu.make_async_copy(srcsA[i], stagA.at[i % 2],
                                      copyA_sems.at[i % 2])
            d.start()
            dmasA[i % 2] = d

        def finishA(i):
            dmasA[i % 2].wait()
            storeA(i, i % 2)

        def startB(b):
            hbm = k_hbm if b < HL else v_hbm
            d = pltpu.make_async_copy(hbm.at[:, b % HL, :],
                                      stagB.at[b % 2],
                                      copyB_sems.at[b % 2])
            d.start()
            dmasB[b % 2] = d

        def finishB(b):
            dmasB[b % 2].wait()
            dst = k_ref if b < HL else v_ref
            dst[b % HL] = stagB[b % 2].astype(jnp.bfloat16)

        startA(0)
        startA(1)
        pl.semaphore_wait(barrier_sem, 3)
        finishA(0)
        startA(2)
        finishA(1)
        startB(0)
        startB(1)
        for j8 in range(SQ // QB):
            if j8 == 2:
                finishA(2)
                startA(3)
            rows = pl.ds(j8 * QB, QB)
            q_ref[rows, :] = (lax.dot_general(
                x_ref[rows, :], wq_ref[...], (((1,), (0,)), ((), ())),
                preferred_element_type=jnp.float32)
                * QSCALE).astype(jnp.bfloat16)
            for t in range(2):
                b = 2 * j8 + t
                finishB(b)
                if b + 2 < 2 * HL:
                    startB(b + 2)
        finishA(3)

        def compute_quarter(qtr):
            for j in range(QROWS // QB):
                row = qtr * QROWS + j * QB
                rows = pl.ds(row, QB)
                s = jnp.minimum(jnp.maximum(row - 128, 0), SKV - KW)
                s = (s // 128) * 128
                qi = lax.broadcasted_iota(jnp.int32, (QB, KW), 0) + row
                kj = lax.broadcasted_iota(jnp.int32, (QB, KW), 1) + s
                maskbias = jnp.where(jnp.abs(qi - kj) <= WIN, 0.0, -1e30)
                for h in range(HL):
                    hcols = pl.ds(h * DH, DH)
                    qblk = q_ref[rows, hcols]
                    kwin = k_ref[h, pl.ds(s, KW), :]
                    vwin = v_ref[h, pl.ds(s, KW), :]
                    scores = lax.dot_general(
                        qblk, kwin, (((1,), (1,)), ((), ())),
                        preferred_element_type=jnp.float32)
                    w = jnp.exp2(scores + maskbias)
                    recip = 1.0 / jnp.sum(w, axis=1, keepdims=True)
                    ctx_blk = lax.dot_general(
                        w.astype(jnp.bfloat16), vwin, (((1,), (0,)), ((), ())),
                        preferred_element_type=jnp.float32) * recip
                    ctx_ref[rows, hcols] = ctx_blk.astype(jnp.bfloat16)
                out_ref[0, rows, :] = lax.dot_general(
                    ctx_ref[rows, :], wo_ref[...], (((1,), (0,)), ((), ())),
                    preferred_element_type=jnp.float32)

        def qmod(c):
            return lax.rem(my_pos + c + 2 * N_DEV, N_DEV)

        def rowQ(q):
            return pl.ds(q * QROWS, QROWS)

        def rowA(q):
            return pl.ds(q * QROWS, QB)

        def rowB(q):
            return pl.ds(q * QROWS + QB, QB)

        dr_rdmas = []
        for r in (1, 2, 3):
            qtr = qmod(r)
            compute_quarter(qtr)
            dr_send[r - 1] = out_ref[0, rowQ(qtr), :].astype(jnp.bfloat16)
            rdma = pltpu.make_async_remote_copy(
                src_ref=dr_send.at[r - 1],
                dst_ref=dr_recv.at[3 - r],
                send_sem=drs_sems.at[r - 1],
                recv_sem=drr_sems.at[3 - r],
                device_id=(qmod(r),),
                device_id_type=pl.DeviceIdType.MESH,
            )
            rdma.start()
            dr_rdmas.append(rdma)

        compute_quarter(qmod(0))
        for rdma in dr_rdmas:
            rdma.wait_send()
        for s in range(3):
            pltpu.make_async_remote_copy(
                src_ref=dr_send.at[s], dst_ref=dr_recv.at[s],
                send_sem=drs_sems.at[s], recv_sem=drr_sems.at[s],
                device_id=(right,), device_id_type=pl.DeviceIdType.MESH,
            ).wait_recv()
        for j in range(QROWS // QB):
            rows = pl.ds(qmod(0) * QROWS + j * QB, QB)
            srows = pl.ds(j * QB, QB)
            acc = out_ref[0, rows, :]
            for s in range(3):
                acc = acc + dr_recv[s, srows, :].astype(jnp.float32)
            out_ref[0, rows, :] = acc

        def copy(src, dst, s_sems, r_sems, idx, dev):
            return pltpu.make_async_remote_copy(
                src_ref=src, dst_ref=dst,
                send_sem=s_sems.at[idx], recv_sem=r_sems.at[idx],
                device_id=(dev,), device_id_type=pl.DeviceIdType.MESH)

        started = []
        for u in range(2):
            baseA = qmod(0) * QROWS + u * HQB
            baseB = qmod(0) * QROWS + QB + u * HQB
            ag_sendR[u] = out_ref[0, pl.ds(baseA, HQB), :].astype(jnp.bfloat16)
            ag_sendL[u] = out_ref[0, pl.ds(baseB, HQB), :].astype(jnp.bfloat16)
        for u in range(2):
            dR = copy(ag_sendR.at[u], ag_recvR.at[0, u],
                      sendR_sems, recvR_sems, u, right)
            dL = copy(ag_sendL.at[u], ag_recvL.at[0, u],
                      sendL_sems, recvL_sems, u, left)
            dR.start()
            dL.start()
            started += [dR, dL]
        for t in range(N_DEV - 1):
            qR = qmod(-1 - t)
            qL = qmod(1 + t)
            for u in range(2):
                copy(ag_sendR.at[u], ag_recvR.at[t, u],
                     sendR_sems, recvR_sems, 2 * t + u, right).wait_recv()
                copy(ag_sendL.at[u], ag_recvL.at[t, u],
                     sendL_sems, recvL_sems, 2 * t + u, left).wait_recv()
                if t < N_DEV - 2:
                    fR = copy(ag_recvR.at[t, u], ag_recvR.at[t + 1, u],
                              sendR_sems, recvR_sems, 2 * (t + 1) + u, right)
                    fL = copy(ag_recvL.at[t, u], ag_recvL.at[t + 1, u],
                              sendL_sems, recvL_sems, 2 * (t + 1) + u, left)
                    fR.start()
                    fL.start()
                    started += [fR, fL]
                out_ref[0, pl.ds(qR * QROWS + u * HQB, HQB), :] = (
                    ag_recvR[t, u].astype(jnp.float32))
                out_ref[0, pl.ds(qL * QROWS + QB + u * HQB, HQB), :] = (
                    ag_recvL[t, u].astype(jnp.float32))
        for d in started:
            d.wait_send()

        @functools.partial(pl.run_scoped, sem2=pltpu.SemaphoreType.REGULAR)
        def _(sem2):
            for nbr in (left, right, diag):
                pl.semaphore_signal(sem2, inc=1, device_id=(nbr,),
                                    device_id_type=pl.DeviceIdType.MESH)
            pl.semaphore_wait(sem2, 3)

    return pl.pallas_call(
        body,
        out_shape=jax.ShapeDtypeStruct((1, SQ, DM), jnp.float32),
        in_specs=[pl.BlockSpec(memory_space=pl.ANY)] * 5,
        out_specs=pl.BlockSpec(memory_space=pltpu.VMEM),
        scratch_shapes=[
            pltpu.VMEM((2, 1024, DM), jnp.float32),
            pltpu.VMEM((2, SKV, DH), jnp.float32),
            pltpu.VMEM((SQ, DM), jnp.bfloat16),
            pltpu.VMEM((HL, SKV, DH), jnp.bfloat16),
            pltpu.VMEM((HL, SKV, DH), jnp.bfloat16),
            pltpu.VMEM((DM, DL), jnp.bfloat16),
            pltpu.VMEM((DL, DM), jnp.bfloat16),
            pltpu.VMEM((SQ, DL), jnp.bfloat16),
            pltpu.VMEM((SQ, DL), jnp.bfloat16),
            pltpu.VMEM((3, QROWS, DM), jnp.bfloat16),
            pltpu.VMEM((3, QROWS, DM), jnp.bfloat16),
            pltpu.VMEM((2, HQB, DM), jnp.bfloat16),
            pltpu.VMEM((3, 2, HQB, DM), jnp.bfloat16),
            pltpu.VMEM((2, HQB, DM), jnp.bfloat16),
            pltpu.VMEM((3, 2, HQB, DM), jnp.bfloat16),
            pltpu.SemaphoreType.DMA((2,)),
            pltpu.SemaphoreType.DMA((2,)),
            pltpu.SemaphoreType.DMA((3,)),
            pltpu.SemaphoreType.DMA((3,)),
            pltpu.SemaphoreType.DMA((6,)),
            pltpu.SemaphoreType.DMA((6,)),
            pltpu.SemaphoreType.DMA((6,)),
            pltpu.SemaphoreType.DMA((6,)),
        ],
        compiler_params=pltpu.CompilerParams(
            collective_id=0,
            vmem_limit_bytes=100 * 1024 * 1024,
        ),
    )(xb, Wq, kb, vb, Wo)
